# baseline (speedup 1.0000x reference)
import functools

import jax
import jax.numpy as jnp
from jax import lax
from jax.experimental import pallas as pl
from jax.experimental.pallas import tpu as pltpu
from jax.experimental.pallas import tpu_sc as plsc

_VOCAB = 100000
_DIM = 64
_B = 1024
_L = 20
_VT = 1024
_NV = (_VOCAB + _VT - 1) // _VT
_VPAD = _NV * _VT


def _sc_embed_mean(table, idx_flat):
    nw = 32
    per_w = _B // nw
    mesh = plsc.VectorSubcoreMesh(core_axis_name="c", subcore_axis_name="s")

    @functools.partial(
        pl.kernel,
        out_type=jax.ShapeDtypeStruct((_B, _DIM), jnp.float32),
        mesh=mesh,
        scratch_types=[
            pltpu.VMEM((per_w * _L,), jnp.int32),
            pltpu.VMEM((per_w * _L, _DIM), jnp.float32),
            pltpu.VMEM((per_w, _DIM), jnp.float32),
            pltpu.SemaphoreType.DMA,
        ],
        compiler_params=pltpu.CompilerParams(use_tc_tiling_on_sc=False),
    )
    def k(table_hbm, idx_hbm, out_hbm, idx_v, rows_v, acc_v, sem):
        wid = lax.axis_index("s") * 2 + lax.axis_index("c")
        base = wid * per_w
        pltpu.sync_copy(idx_hbm.at[pl.ds(base * _L, per_w * _L)], idx_v)
        pltpu.async_copy(table_hbm.at[idx_v], rows_v, sem).wait()

        @pl.loop(0, per_w)
        def _(b):
            r0 = b * _L
            for d in range(_DIM // 16):
                sl = pl.ds(d * 16, 16)
                acc = rows_v[r0, sl]
                for l in range(1, _L):
                    acc = acc + rows_v[r0 + l, sl]
                acc_v[b, sl] = acc * (1.0 / _L)

        pltpu.sync_copy(acc_v, out_hbm.at[pl.ds(base, per_w)])

    return k(table, idx_flat)


def _tc_body(emb_ref, w_ref, b_ref, out_ref, c_scr):
    v = pl.program_id(0)
    emb = emb_ref[...]

    def logits_tile(t):
        wc = w_ref[:, pl.ds(t * _VT, _VT)]
        acc = lax.dot_general(
            emb, wc, (((1,), (0,)), ((), ())),
            preferred_element_type=jnp.float32,
        )
        return acc + b_ref[:, pl.ds(t * _VT, _VT)]

    @pl.when(v == 0)
    def _():
        def step(i, carry):
            m, s = carry
            lg = logits_tile(i)
            mn = jnp.maximum(m, jnp.max(lg, axis=1, keepdims=True))
            s = s * jnp.exp(m - mn) + jnp.sum(
                jnp.exp(lg - mn), axis=1, keepdims=True)
            return mn, s

        m0 = jnp.full((_B, 1), -1e30, jnp.float32)
        s0 = jnp.zeros((_B, 1), jnp.float32)
        m, s = lax.fori_loop(0, _NV, step, (m0, s0))
        c = m + jnp.log(s)
        c_scr[...] = jnp.broadcast_to(c, (_B, 128))

    c = jnp.max(c_scr[...], axis=1, keepdims=True)
    out_ref[...] = logits_tile(v) - c


def _tc_linear_logsoftmax(emb_bf, w_bf_t, bias_pad):
    return pl.pallas_call(
        _tc_body,
        grid=(_NV,),
        in_specs=[
            pl.BlockSpec((_B, _DIM), lambda v: (0, 0)),
            pl.BlockSpec((_DIM, _VPAD), lambda v: (0, 0)),
            pl.BlockSpec((1, _VPAD), lambda v: (0, 0)),
        ],
        out_specs=pl.BlockSpec((_B, _VT), lambda v: (0, v)),
        out_shape=jax.ShapeDtypeStruct((_B, _VOCAB), jnp.float32),
        scratch_shapes=[pltpu.VMEM((_B, 128), jnp.float32)],
    )(emb_bf, w_bf_t, bias_pad)


def kernel(input_idx, embedding_weight, linear1_weight, linear1_bias):
    idx_flat = input_idx.reshape(-1).astype(jnp.int32)
    embeds = _sc_embed_mean(embedding_weight, idx_flat)
    emb_bf = embeds.astype(jnp.bfloat16)
    w_bf_t = jnp.pad(
        linear1_weight.astype(jnp.bfloat16).T, ((0, 0), (0, _VPAD - _VOCAB)))
    bias_pad = jnp.pad(
        linear1_bias, (0, _VPAD - _VOCAB), constant_values=-1e30
    ).reshape(1, _VPAD)
    return _tc_linear_logsoftmax(emb_bf, w_bf_t, bias_pad)
